# Initial kernel scaffold; baseline (speedup 1.0000x reference)
import jax
import jax.numpy as jnp
from jax.experimental import pallas as pl


def kernel(x, edge_index, edge_weights, batch, W_pre1, b_pre1, W_post1, b_post1, W_lin1, b_lin1, W_pre2, b_pre2, W_post2, b_post2, W_lin2, b_lin2, W_out, b_out):
    raise NotImplementedError("write your pallas kernel here")


# speedup vs baseline: 1.1837x; 1.1837x over previous
import functools
import numpy as np
import jax
import jax.numpy as jnp
from jax.experimental import pallas as pl
from jax.experimental.pallas import tpu as pltpu

N = 10000
F = 128
_AVG_LOG = float(np.log(33.0))

_POST_BLOCK = 1000


def _post_kernel(x_ref, agg_ref, sca_ref, g_ref, gb_ref, o_ref):
    a = sca_ref[:, 0:1]
    b = sca_ref[:, 1:2]
    agg = agg_ref[:]
    z = jnp.concatenate([x_ref[:], agg, agg * a, agg * b], axis=1)
    o = jnp.dot(z, g_ref[:], preferred_element_type=jnp.float32) + gb_ref[:]
    o_ref[:] = jnp.maximum(o, 0.0)


def _post_matmul_relu(x, agg, sca, G, gb):
    nb = N // _POST_BLOCK
    return pl.pallas_call(
        _post_kernel,
        grid=(nb,),
        in_specs=[
            pl.BlockSpec((_POST_BLOCK, F), lambda i: (i, 0)),
            pl.BlockSpec((_POST_BLOCK, 4 * F), lambda i: (i, 0)),
            pl.BlockSpec((_POST_BLOCK, 2), lambda i: (i, 0)),
            pl.BlockSpec((13 * F, F), lambda i: (0, 0)),
            pl.BlockSpec((F,), lambda i: (0,)),
        ],
        out_specs=pl.BlockSpec((_POST_BLOCK, F), lambda i: (i, 0)),
        out_shape=jax.ShapeDtypeStruct((N, F), jnp.float32),
    )(x, agg, sca, G, gb)


def _segment_stats(xs, src, dst, cnt, cnt_c, xd):
    rows = xs[src]
    S1 = jax.ops.segment_sum(rows, dst, num_segments=N)
    S2 = jax.ops.segment_sum(rows * rows, dst, num_segments=N)
    MN = jax.ops.segment_min(rows, dst, num_segments=N)
    MX = jax.ops.segment_max(rows, dst, num_segments=N)
    cc = cnt_c[:, None]
    cpos = cnt[:, None] > 0
    mean = (cnt[:, None] * xd + S1) / cc
    msq = (cnt[:, None] * xd * xd + 2.0 * xd * S1 + S2) / cc
    var = jnp.maximum(msq - mean * mean, 0.0)
    std = jnp.sqrt(var + 1e-5)
    mn = jnp.where(cpos, xd + MN, 0.0)
    mx = jnp.where(cpos, xd + MX, 0.0)
    return jnp.concatenate([mean, mn, mx, std], axis=-1)


def kernel(x, edge_index, edge_weights, batch, W_pre1, b_pre1, W_post1, b_post1,
           W_lin1, b_lin1, W_pre2, b_pre2, W_post2, b_post2, W_lin2, b_lin2,
           W_out, b_out):
    src = edge_index[0]
    dst = edge_index[1]

    G1 = W_post1 @ W_lin1
    g1 = b_post1 @ W_lin1 + b_lin1
    G2 = W_post2 @ W_lin2
    g2 = b_post2 @ W_lin2 + b_lin2

    cnt = jax.ops.segment_sum(jnp.ones((src.shape[0],), jnp.float32), dst,
                              num_segments=N)
    cnt_c = jnp.maximum(cnt, 1.0)
    lg = jnp.log(cnt_c + 1.0)
    a_sc = lg / _AVG_LOG
    b_sc = _AVG_LOG / lg
    sca = jnp.stack([a_sc, b_sc], axis=1)

    xd1 = x @ W_pre1[:F] + b_pre1
    xs1 = x @ W_pre1[F:]
    agg1 = _segment_stats(xs1, src, dst, cnt, cnt_c, xd1)
    h1 = _post_matmul_relu(x, agg1, sca, G1, g1)

    xd2 = h1 @ W_pre2[:F] + b_pre2
    xs2 = h1 @ W_pre2[F:]
    agg2 = _segment_stats(xs2, src, dst, cnt, cnt_c, xd2)

    w = W_out[:, 0]
    zbar = jnp.concatenate([
        w @ h1,
        w @ agg2,
        (w * a_sc) @ agg2,
        (w * b_sc) @ agg2,
    ])
    out = zbar @ G2 + jnp.sum(w) * g2 + b_out
    return out[None, :]


# speedup vs baseline: 1.5663x; 1.3232x over previous
import functools
import numpy as np
import jax
import jax.numpy as jnp
from jax import lax
from jax.experimental import pallas as pl
from jax.experimental.pallas import tpu as pltpu
from jax.experimental.pallas import tpu_sc as plsc

N = 10000
E = 320000
F = 128
_AVG_LOG = float(np.log(33.0))

_NW = 32
_NCHUNK = 64
_M = 160
_NPAD = _NCHUNK * _M
_BE = 2000
_NB = E // _BE
_GV = 5
_NG = _BE // (16 * _GV)
_G = 256
_STAGE = _G + 16 * _GV


def _sc_segment_stats(xs, src, dst):
    mesh = plsc.VectorSubcoreMesh(core_axis_name="c", subcore_axis_name="s")

    @functools.partial(
        pl.kernel,
        mesh=mesh,
        compiler_params=pltpu.CompilerParams(needs_layout_passes=False),
        out_type=(
            jax.ShapeDtypeStruct((_NPAD * 128,), jnp.float32),
            jax.ShapeDtypeStruct((_NPAD * 128,), jnp.float32),
            jax.ShapeDtypeStruct((_NPAD * 128,), jnp.float32),
            jax.ShapeDtypeStruct((_NPAD * 128,), jnp.float32),
            jax.ShapeDtypeStruct((_NPAD,), jnp.float32),
        ),
        scratch_types=[
            pltpu.VMEM((_M * 128,), jnp.float32),
            pltpu.VMEM((_M * 128,), jnp.float32),
            pltpu.VMEM((_M * 128,), jnp.float32),
            pltpu.VMEM((_M * 128,), jnp.float32),
            pltpu.VMEM((_M + 16,), jnp.float32),
            pltpu.VMEM((_BE,), jnp.int32),
            pltpu.VMEM((_BE,), jnp.int32),
            pltpu.VMEM((_STAGE + 32,), jnp.int32),
            pltpu.VMEM((_G,), jnp.int32),
            pltpu.VMEM((_G, 128), jnp.float32),
            pltpu.VMEM((32,), jnp.int32),
            pltpu.SemaphoreType.DMA,
        ],
    )
    def k(xs_hbm, src_hbm, dst_hbm, s_out, q_out, mn_out, mx_out, cnt_out,
          accS, accQ, accMN, accMX, cntv, dbuf, sbuf, sstg, gidx, rows, shsc,
          sem):
        wid = lax.axis_index("s") * 2 + lax.axis_index("c")
        zero16 = jnp.zeros((16,), jnp.float32)
        inf16 = jnp.full((16,), jnp.inf, jnp.float32)
        ones16 = jnp.ones((16,), jnp.float32)
        smask16 = jnp.full((16,), 0x3FFFF, jnp.int32)

        for t in range((_STAGE + 32) // 16):
            sstg[pl.ds(t * 16, 16)] = jnp.zeros((16,), jnp.int32)
        shsc[pl.ds(0, 16)] = jnp.zeros((16,), jnp.int32)

        def rmw(count):
            def edge_body(j, _):
                packed = sstg[pl.ds(j, 16)][0]
                base = (packed >> 18) * 128
                for c in range(8):
                    r = rows[j, pl.ds(c * 16, 16)]
                    asl = pl.ds(base + c * 16, 16)
                    plsc.addupdate(accS.at[asl], r)
                    plsc.addupdate(accQ.at[asl], r * r)
                    mv = accMN[asl]
                    accMN[asl] = jnp.minimum(mv, r)
                    xv = accMX[asl]
                    accMX[asl] = jnp.maximum(xv, r)
                return 0
            lax.fori_loop(0, count, edge_body, 0)

        def fire_gather():
            for t in range(_G // 16):
                gidx[pl.ds(t * 16, 16)] = sstg[pl.ds(t * 16, 16)] & smask16
            pltpu.async_copy(xs_hbm.at[gidx], rows, sem).wait()

        for ci in range(2):
            chunk = wid * 2 + ci
            lo = chunk * _M
            hi = lo + _M

            def zero_body(j, _):
                sl = pl.ds(j * 16, 16)
                accS[sl] = zero16
                accQ[sl] = zero16
                accMN[sl] = inf16
                accMX[sl] = -inf16
                return 0
            lax.fori_loop(0, _M * 128 // 16, zero_body, 0)
            for t in range((_M + 16) // 16):
                cntv[pl.ds(t * 16, 16)] = zero16

            def drain(off):
                fire_gather()
                rmw(_G)
                for t in range(_GV):
                    sstg[pl.ds(t * 16, 16)] = sstg[pl.ds(_G + t * 16, 16)]
                return off - _G

            def batch_body(ib, off):
                ebase = ib * _BE
                pltpu.sync_copy(dst_hbm.at[pl.ds(ebase, _BE)], dbuf)
                pltpu.sync_copy(src_hbm.at[pl.ds(ebase, _BE)], sbuf)

                def group_body(ig, off):
                    gbase = ig * (16 * _GV)
                    for v in range(_GV):
                        sl = pl.ds(gbase + v * 16, 16)
                        d = dbuf[sl]
                        s = sbuf[sl]
                        m = (d >= lo) & (d < hi)
                        packed = ((d - lo) << 18) | s
                        p = jnp.where(m, 1, 0)
                        for sh in (1, 2, 4, 8):
                            shsc[pl.ds(16, 16)] = p
                            p = p + shsc[pl.ds(16 - sh, 16)]
                        pos = jnp.where(m, p + (off - 1), _STAGE + 16)
                        plsc.store_scatter(sstg, [pos], packed)
                        cpos = jnp.where(m, d - lo, _M)
                        plsc.addupdate_scatter(cntv, [cpos], ones16)
                        off = off + p[15]
                    return lax.cond(off >= _G, drain, lambda o: o, off)

                return lax.fori_loop(0, _NG, group_body, off)

            off = lax.fori_loop(0, _NB, batch_body, jnp.int32(0))

            fire_gather()
            rmw(off)

            pltpu.sync_copy(accS, s_out.at[pl.ds(lo * 128, _M * 128)])
            pltpu.sync_copy(accQ, q_out.at[pl.ds(lo * 128, _M * 128)])
            pltpu.sync_copy(accMN, mn_out.at[pl.ds(lo * 128, _M * 128)])
            pltpu.sync_copy(accMX, mx_out.at[pl.ds(lo * 128, _M * 128)])
            pltpu.sync_copy(cntv.at[pl.ds(0, _M)], cnt_out.at[pl.ds(lo, _M)])

    return k(xs, src, dst)


_POST_BLOCK = 1000


def _epilogue(x_blk, xd, S, Q, MN, MX, cnt_col):
    cc = jnp.maximum(cnt_col, 1.0)
    cpos = cnt_col > 0
    mean = (cnt_col * xd + S) / cc
    msq = (cnt_col * xd * xd + 2.0 * xd * S + Q) / cc
    var = jnp.maximum(msq - mean * mean, 0.0)
    std = jnp.sqrt(var + 1e-5)
    mn = jnp.where(cpos, xd + MN, 0.0)
    mx = jnp.where(cpos, xd + MX, 0.0)
    agg = jnp.concatenate([mean, mn, mx, std], axis=-1)
    lg = jnp.log(cc + 1.0)
    a_sc = lg / _AVG_LOG
    b_sc = _AVG_LOG / lg
    return jnp.concatenate([x_blk, agg, agg * a_sc, agg * b_sc], axis=1)


def _pre_kernel(x_ref, wd_ref, ws_ref, bd_ref, o1_ref, o2_ref):
    x = x_ref[:]
    o1_ref[:] = jnp.dot(x, wd_ref[:], preferred_element_type=jnp.float32) + bd_ref[:]
    o2_ref[:] = jnp.dot(x, ws_ref[:], preferred_element_type=jnp.float32)


def _pre_matmuls(x, Wd, Ws, bd):
    nb = N // _POST_BLOCK
    return pl.pallas_call(
        _pre_kernel,
        grid=(nb,),
        in_specs=[
            pl.BlockSpec((_POST_BLOCK, F), lambda i: (i, 0)),
            pl.BlockSpec((F, F), lambda i: (0, 0)),
            pl.BlockSpec((F, F), lambda i: (0, 0)),
            pl.BlockSpec((F,), lambda i: (0,)),
        ],
        out_specs=[
            pl.BlockSpec((_POST_BLOCK, F), lambda i: (i, 0)),
            pl.BlockSpec((_POST_BLOCK, F), lambda i: (i, 0)),
        ],
        out_shape=[
            jax.ShapeDtypeStruct((N, F), jnp.float32),
            jax.ShapeDtypeStruct((N, F), jnp.float32),
        ],
    )(x, Wd, Ws, bd)


def _post1_kernel(x_ref, xd_ref, s_ref, q_ref, mn_ref, mx_ref, cnt_ref,
                  g_ref, gb_ref, o_ref):
    z = _epilogue(x_ref[:], xd_ref[:], s_ref[:], q_ref[:], mn_ref[:],
                  mx_ref[:], cnt_ref[:])
    o = jnp.dot(z, g_ref[:], preferred_element_type=jnp.float32) + gb_ref[:]
    o_ref[:] = jnp.maximum(o, 0.0)


def _post1(x, xd, S, Q, MN, MX, cnt, G1, g1):
    nb = N // _POST_BLOCK
    blk = lambda w: pl.BlockSpec((_POST_BLOCK, w), lambda i: (i, 0))
    return pl.pallas_call(
        _post1_kernel,
        grid=(nb,),
        in_specs=[
            blk(F), blk(F), blk(F), blk(F), blk(F), blk(F),
            pl.BlockSpec((_POST_BLOCK, 1), lambda i: (i, 0)),
            pl.BlockSpec((13 * F, F), lambda i: (0, 0)),
            pl.BlockSpec((F,), lambda i: (0,)),
        ],
        out_specs=blk(F),
        out_shape=jax.ShapeDtypeStruct((N, F), jnp.float32),
    )(x, xd, S, Q, MN, MX, cnt, G1, g1)


def _post2_kernel(h_ref, xd_ref, s_ref, q_ref, mn_ref, mx_ref, cnt_ref,
                  w_ref, o_ref):
    i = pl.program_id(0)
    z = _epilogue(h_ref[:], xd_ref[:], s_ref[:], q_ref[:], mn_ref[:],
                  mx_ref[:], cnt_ref[:])
    part = jnp.dot(w_ref[:].reshape(1, _POST_BLOCK), z,
                   preferred_element_type=jnp.float32)

    @pl.when(i == 0)
    def _():
        o_ref[:] = jnp.zeros_like(o_ref)
    o_ref[:] += part


def _post2_zbar(h1, xd, S, Q, MN, MX, cnt, w):
    nb = N // _POST_BLOCK
    blk = lambda wdt: pl.BlockSpec((_POST_BLOCK, wdt), lambda i: (i, 0))
    return pl.pallas_call(
        _post2_kernel,
        grid=(nb,),
        in_specs=[
            blk(F), blk(F), blk(F), blk(F), blk(F), blk(F),
            pl.BlockSpec((_POST_BLOCK, 1), lambda i: (i, 0)),
            pl.BlockSpec((_POST_BLOCK, 1), lambda i: (i, 0)),
        ],
        out_specs=pl.BlockSpec((1, 13 * F), lambda i: (0, 0)),
        out_shape=jax.ShapeDtypeStruct((1, 13 * F), jnp.float32),
    )(h1, xd, S, Q, MN, MX, cnt, w)


def kernel(x, edge_index, edge_weights, batch, W_pre1, b_pre1, W_post1, b_post1,
           W_lin1, b_lin1, W_pre2, b_pre2, W_post2, b_post2, W_lin2, b_lin2,
           W_out, b_out):
    src = edge_index[0]
    dst = edge_index[1]

    G1 = W_post1 @ W_lin1
    g1 = b_post1 @ W_lin1 + b_lin1
    G2 = W_post2 @ W_lin2
    g2 = b_post2 @ W_lin2 + b_lin2

    xd1, xs1 = _pre_matmuls(x, W_pre1[:F], W_pre1[F:], b_pre1)
    S1, Q1, MN1, MX1, cnt = _sc_segment_stats(xs1, src, dst)
    S1 = S1.reshape(_NPAD, 128)[:N]
    Q1 = Q1.reshape(_NPAD, 128)[:N]
    MN1 = MN1.reshape(_NPAD, 128)[:N]
    MX1 = MX1.reshape(_NPAD, 128)[:N]
    cnt_col = cnt[:N, None]
    h1 = _post1(x, xd1, S1, Q1, MN1, MX1, cnt_col, G1, g1)

    xd2, xs2 = _pre_matmuls(h1, W_pre2[:F], W_pre2[F:], b_pre2)
    S2, Q2, MN2, MX2, _ = _sc_segment_stats(xs2, src, dst)
    S2 = S2.reshape(_NPAD, 128)[:N]
    Q2 = Q2.reshape(_NPAD, 128)[:N]
    MN2 = MN2.reshape(_NPAD, 128)[:N]
    MX2 = MX2.reshape(_NPAD, 128)[:N]

    zbar = _post2_zbar(h1, xd2, S2, Q2, MN2, MX2, cnt_col, W_out)[0]
    out = zbar @ G2 + jnp.sum(W_out) * g2 + b_out
    return out[None, :]


# speedup vs baseline: 2.4254x; 1.5485x over previous
import functools
import numpy as np
import jax
import jax.numpy as jnp
from jax import lax
from jax.experimental import pallas as pl
from jax.experimental.pallas import tpu as pltpu
from jax.experimental.pallas import tpu_sc as plsc

N = 10000
E = 320000
F = 128
_AVG_LOG = float(np.log(33.0))

_NW = 32
_NCHUNK = 64
_M = 160
_NPAD = _NCHUNK * _M
_BE = 2000
_NB = E // _BE
_GV = 5
_NG = _BE // (16 * _GV)
_G = 256
_STAGE = _G + 16 * _GV
_ECAP = E + _G

_SC_MESH_KW = dict(
    compiler_params=pltpu.CompilerParams(needs_layout_passes=False),
)


def _acc_outs():
    return (
        jax.ShapeDtypeStruct((_NPAD * 128,), jnp.float32),
        jax.ShapeDtypeStruct((_NPAD * 128,), jnp.float32),
        jax.ShapeDtypeStruct((_NPAD * 128,), jnp.float32),
        jax.ShapeDtypeStruct((_NPAD * 128,), jnp.float32),
    )


def _make_rmw(sstg, rows, accS, accQ, accMN, accMX):
    def rmw(count):
        def edge_body(j, _):
            packed = sstg[pl.ds(j, 16)][0]
            base = (packed >> 18) * 128
            for c in range(8):
                r = rows[j, pl.ds(c * 16, 16)]
                asl = pl.ds(base + c * 16, 16)
                plsc.addupdate(accS.at[asl], r)
                plsc.addupdate(accQ.at[asl], r * r)
                mv = accMN[asl]
                accMN[asl] = jnp.minimum(mv, r)
                xv = accMX[asl]
                accMX[asl] = jnp.maximum(xv, r)
            return 0
        lax.fori_loop(0, count, edge_body, 0)
    return rmw


def _make_fire_gather(xs_hbm, sstg, gidx, rows, sem):
    smask16 = jnp.full((16,), 0x3FFFF, jnp.int32)
    nmax16 = jnp.full((16,), N - 1, jnp.int32)

    def fire_gather():
        for t in range(_G // 16):
            gidx[pl.ds(t * 16, 16)] = jnp.minimum(
                sstg[pl.ds(t * 16, 16)] & smask16, nmax16)
        pltpu.async_copy(xs_hbm.at[gidx], rows, sem).wait()
    return fire_gather


def _make_zero_accs(accS, accQ, accMN, accMX):
    zero16 = jnp.zeros((16,), jnp.float32)
    inf16 = jnp.full((16,), jnp.inf, jnp.float32)

    def zero_accs():
        def zero_body(j, _):
            sl = pl.ds(j * 16, 16)
            accS[sl] = zero16
            accQ[sl] = zero16
            accMN[sl] = inf16
            accMX[sl] = -inf16
            return 0
        lax.fori_loop(0, _M * 128 // 16, zero_body, 0)
    return zero_accs


def _writeback(accS, accQ, accMN, accMX, s_out, q_out, mn_out, mx_out, lo):
    pltpu.sync_copy(accS, s_out.at[pl.ds(lo * 128, _M * 128)])
    pltpu.sync_copy(accQ, q_out.at[pl.ds(lo * 128, _M * 128)])
    pltpu.sync_copy(accMN, mn_out.at[pl.ds(lo * 128, _M * 128)])
    pltpu.sync_copy(accMX, mx_out.at[pl.ds(lo * 128, _M * 128)])


def _sc_stats_partition(xs, src, dst):
    mesh = plsc.VectorSubcoreMesh(core_axis_name="c", subcore_axis_name="s")

    @functools.partial(
        pl.kernel,
        mesh=mesh,
        out_type=_acc_outs() + (
            jax.ShapeDtypeStruct((_NPAD,), jnp.float32),
            jax.ShapeDtypeStruct((_NCHUNK, _ECAP), jnp.int32),
            jax.ShapeDtypeStruct((_NCHUNK * 16,), jnp.int32),
        ),
        scratch_types=[
            pltpu.VMEM((_M * 128,), jnp.float32),
            pltpu.VMEM((_M * 128,), jnp.float32),
            pltpu.VMEM((_M * 128,), jnp.float32),
            pltpu.VMEM((_M * 128,), jnp.float32),
            pltpu.VMEM((_M + 16,), jnp.float32),
            pltpu.VMEM((_BE,), jnp.int32),
            pltpu.VMEM((_BE,), jnp.int32),
            pltpu.VMEM((_STAGE + 32,), jnp.int32),
            pltpu.VMEM((_G,), jnp.int32),
            pltpu.VMEM((_G, 128), jnp.float32),
            pltpu.VMEM((32,), jnp.int32),
            pltpu.VMEM((16,), jnp.int32),
            pltpu.SemaphoreType.DMA,
        ],
        **_SC_MESH_KW,
    )
    def k(xs_hbm, src_hbm, dst_hbm, s_out, q_out, mn_out, mx_out, cnt_out,
          parts_out, pcnt_out, accS, accQ, accMN, accMX, cntv, dbuf, sbuf,
          sstg, gidx, rows, shsc, tebuf, sem):
        wid = lax.axis_index("s") * 2 + lax.axis_index("c")
        zero16 = jnp.zeros((16,), jnp.float32)
        ones16 = jnp.ones((16,), jnp.float32)
        rmw = _make_rmw(sstg, rows, accS, accQ, accMN, accMX)
        fire_gather = _make_fire_gather(xs_hbm, sstg, gidx, rows, sem)
        zero_accs = _make_zero_accs(accS, accQ, accMN, accMX)

        for t in range((_STAGE + 32) // 16):
            sstg[pl.ds(t * 16, 16)] = jnp.zeros((16,), jnp.int32)
        shsc[pl.ds(0, 16)] = jnp.zeros((16,), jnp.int32)

        for ci in range(2):
            chunk = wid * 2 + ci
            lo = chunk * _M
            hi = lo + _M
            zero_accs()
            for t in range((_M + 16) // 16):
                cntv[pl.ds(t * 16, 16)] = zero16

            def drain(carry):
                off, nd = carry
                pltpu.sync_copy(sstg.at[pl.ds(0, _G)],
                                parts_out.at[chunk, pl.ds(nd * _G, _G)])
                fire_gather()
                rmw(_G)
                for t in range(_GV):
                    sstg[pl.ds(t * 16, 16)] = sstg[pl.ds(_G + t * 16, 16)]
                return off - _G, nd + 1

            def batch_body(ib, carry):
                ebase = ib * _BE
                pltpu.sync_copy(dst_hbm.at[pl.ds(ebase, _BE)], dbuf)
                pltpu.sync_copy(src_hbm.at[pl.ds(ebase, _BE)], sbuf)

                def group_body(ig, carry):
                    off, nd = carry
                    gbase = ig * (16 * _GV)
                    for v in range(_GV):
                        sl = pl.ds(gbase + v * 16, 16)
                        d = dbuf[sl]
                        m = (d >= lo) & (d < hi)
                        pc = plsc.all_reduce_population_count(m)[0]

                        def hit(o):
                            s = sbuf[sl]
                            packed = ((d - lo) << 18) | s
                            p = jnp.where(m, 1, 0)
                            for sh in (1, 2, 4, 8):
                                shsc[pl.ds(16, 16)] = p
                                p = p + shsc[pl.ds(16 - sh, 16)]
                            pos = jnp.where(m, p + (o - 1), _STAGE + 16)
                            plsc.store_scatter(sstg, [pos], packed)
                            cpos = jnp.where(m, d - lo, _M)
                            plsc.addupdate_scatter(cntv, [cpos], ones16)
                            return o + pc

                        off = lax.cond(pc > 0, hit, lambda o: o, off)
                    return lax.cond(off >= _G, drain, lambda c: c, (off, nd))

                return lax.fori_loop(0, _NG, group_body, carry)

            off, nd = lax.fori_loop(0, _NB, batch_body,
                                    (jnp.int32(0), jnp.int32(0)))

            pltpu.sync_copy(sstg.at[pl.ds(0, _G)],
                            parts_out.at[chunk, pl.ds(nd * _G, _G)])
            fire_gather()
            rmw(off)
            te = nd * _G + off
            tebuf[pl.ds(0, 16)] = jnp.full((16,), 1, jnp.int32) * te
            pltpu.sync_copy(tebuf, pcnt_out.at[pl.ds(chunk * 16, 16)])

            _writeback(accS, accQ, accMN, accMX, s_out, q_out, mn_out, mx_out, lo)
            pltpu.sync_copy(cntv.at[pl.ds(0, _M)], cnt_out.at[pl.ds(lo, _M)])

    return k(xs, src, dst)


def _sc_stats_from_parts(xs, parts, pcnt):
    mesh = plsc.VectorSubcoreMesh(core_axis_name="c", subcore_axis_name="s")

    @functools.partial(
        pl.kernel,
        mesh=mesh,
        out_type=_acc_outs(),
        scratch_types=[
            pltpu.VMEM((_M * 128,), jnp.float32),
            pltpu.VMEM((_M * 128,), jnp.float32),
            pltpu.VMEM((_M * 128,), jnp.float32),
            pltpu.VMEM((_M * 128,), jnp.float32),
            pltpu.VMEM((_STAGE + 32,), jnp.int32),
            pltpu.VMEM((_G,), jnp.int32),
            pltpu.VMEM((_G, 128), jnp.float32),
            pltpu.VMEM((_NCHUNK * 16,), jnp.int32),
            pltpu.SemaphoreType.DMA,
        ],
        **_SC_MESH_KW,
    )
    def k(xs_hbm, parts_hbm, pcnt_hbm, s_out, q_out, mn_out, mx_out,
          accS, accQ, accMN, accMX, sstg, gidx, rows, pcv, sem):
        wid = lax.axis_index("s") * 2 + lax.axis_index("c")
        rmw = _make_rmw(sstg, rows, accS, accQ, accMN, accMX)
        fire_gather = _make_fire_gather(xs_hbm, sstg, gidx, rows, sem)
        zero_accs = _make_zero_accs(accS, accQ, accMN, accMX)
        pltpu.sync_copy(pcnt_hbm, pcv)

        for ci in range(2):
            chunk = wid * 2 + ci
            lo = chunk * _M
            zero_accs()
            te = pcv[pl.ds(chunk * 16, 16)][0]
            nfull = te >> 8
            rem = te & (_G - 1)

            def blk(b, _):
                pltpu.sync_copy(parts_hbm.at[chunk, pl.ds(b * _G, _G)],
                                sstg.at[pl.ds(0, _G)])
                fire_gather()
                rmw(_G)
                return 0
            lax.fori_loop(0, nfull, blk, 0)

            pltpu.sync_copy(parts_hbm.at[chunk, pl.ds(nfull * _G, _G)],
                            sstg.at[pl.ds(0, _G)])
            fire_gather()
            rmw(rem)

            _writeback(accS, accQ, accMN, accMX, s_out, q_out, mn_out, mx_out, lo)

    return k(xs, parts, pcnt)


_POST_BLOCK = 1000


def _epilogue(x_blk, xd, S, Q, MN, MX, cnt_col):
    cc = jnp.maximum(cnt_col, 1.0)
    cpos = cnt_col > 0
    mean = (cnt_col * xd + S) / cc
    msq = (cnt_col * xd * xd + 2.0 * xd * S + Q) / cc
    var = jnp.maximum(msq - mean * mean, 0.0)
    std = jnp.sqrt(var + 1e-5)
    mn = jnp.where(cpos, xd + MN, 0.0)
    mx = jnp.where(cpos, xd + MX, 0.0)
    agg = jnp.concatenate([mean, mn, mx, std], axis=-1)
    lg = jnp.log(cc + 1.0)
    a_sc = lg / _AVG_LOG
    b_sc = _AVG_LOG / lg
    return jnp.concatenate([x_blk, agg, agg * a_sc, agg * b_sc], axis=1)


def _pre_kernel(x_ref, wd_ref, ws_ref, bd_ref, o1_ref, o2_ref):
    x = x_ref[:]
    o1_ref[:] = jnp.dot(x, wd_ref[:], preferred_element_type=jnp.float32) + bd_ref[:]
    o2_ref[:] = jnp.dot(x, ws_ref[:], preferred_element_type=jnp.float32)


def _pre_matmuls(x, Wd, Ws, bd):
    nb = N // _POST_BLOCK
    return pl.pallas_call(
        _pre_kernel,
        grid=(nb,),
        in_specs=[
            pl.BlockSpec((_POST_BLOCK, F), lambda i: (i, 0)),
            pl.BlockSpec((F, F), lambda i: (0, 0)),
            pl.BlockSpec((F, F), lambda i: (0, 0)),
            pl.BlockSpec((F,), lambda i: (0,)),
        ],
        out_specs=[
            pl.BlockSpec((_POST_BLOCK, F), lambda i: (i, 0)),
            pl.BlockSpec((_POST_BLOCK, F), lambda i: (i, 0)),
        ],
        out_shape=[
            jax.ShapeDtypeStruct((N, F), jnp.float32),
            jax.ShapeDtypeStruct((N, F), jnp.float32),
        ],
    )(x, Wd, Ws, bd)


def _post1_kernel(x_ref, xd_ref, s_ref, q_ref, mn_ref, mx_ref, cnt_ref,
                  g_ref, gb_ref, o_ref):
    z = _epilogue(x_ref[:], xd_ref[:], s_ref[:], q_ref[:], mn_ref[:],
                  mx_ref[:], cnt_ref[:])
    o = jnp.dot(z, g_ref[:], preferred_element_type=jnp.float32) + gb_ref[:]
    o_ref[:] = jnp.maximum(o, 0.0)


def _post1(x, xd, S, Q, MN, MX, cnt, G1, g1):
    nb = N // _POST_BLOCK
    blk = lambda w: pl.BlockSpec((_POST_BLOCK, w), lambda i: (i, 0))
    return pl.pallas_call(
        _post1_kernel,
        grid=(nb,),
        in_specs=[
            blk(F), blk(F), blk(F), blk(F), blk(F), blk(F),
            pl.BlockSpec((_POST_BLOCK, 1), lambda i: (i, 0)),
            pl.BlockSpec((13 * F, F), lambda i: (0, 0)),
            pl.BlockSpec((F,), lambda i: (0,)),
        ],
        out_specs=blk(F),
        out_shape=jax.ShapeDtypeStruct((N, F), jnp.float32),
    )(x, xd, S, Q, MN, MX, cnt, G1, g1)


def _post2_kernel(h_ref, xd_ref, s_ref, q_ref, mn_ref, mx_ref, cnt_ref,
                  w_ref, o_ref):
    i = pl.program_id(0)
    z = _epilogue(h_ref[:], xd_ref[:], s_ref[:], q_ref[:], mn_ref[:],
                  mx_ref[:], cnt_ref[:])
    part = jnp.dot(w_ref[:].reshape(1, _POST_BLOCK), z,
                   preferred_element_type=jnp.float32)

    @pl.when(i == 0)
    def _():
        o_ref[:] = jnp.zeros_like(o_ref)
    o_ref[:] += part


def _post2_zbar(h1, xd, S, Q, MN, MX, cnt, w):
    nb = N // _POST_BLOCK
    blk = lambda wdt: pl.BlockSpec((_POST_BLOCK, wdt), lambda i: (i, 0))
    return pl.pallas_call(
        _post2_kernel,
        grid=(nb,),
        in_specs=[
            blk(F), blk(F), blk(F), blk(F), blk(F), blk(F),
            pl.BlockSpec((_POST_BLOCK, 1), lambda i: (i, 0)),
            pl.BlockSpec((_POST_BLOCK, 1), lambda i: (i, 0)),
        ],
        out_specs=pl.BlockSpec((1, 13 * F), lambda i: (0, 0)),
        out_shape=jax.ShapeDtypeStruct((1, 13 * F), jnp.float32),
    )(h1, xd, S, Q, MN, MX, cnt, w)


def kernel(x, edge_index, edge_weights, batch, W_pre1, b_pre1, W_post1, b_post1,
           W_lin1, b_lin1, W_pre2, b_pre2, W_post2, b_post2, W_lin2, b_lin2,
           W_out, b_out):
    src = edge_index[0]
    dst = edge_index[1]

    G1 = W_post1 @ W_lin1
    g1 = b_post1 @ W_lin1 + b_lin1
    G2 = W_post2 @ W_lin2
    g2 = b_post2 @ W_lin2 + b_lin2

    xd1, xs1 = _pre_matmuls(x, W_pre1[:F], W_pre1[F:], b_pre1)
    S1, Q1, MN1, MX1, cnt, parts, pcnt = _sc_stats_partition(xs1, src, dst)
    S1 = S1.reshape(_NPAD, 128)[:N]
    Q1 = Q1.reshape(_NPAD, 128)[:N]
    MN1 = MN1.reshape(_NPAD, 128)[:N]
    MX1 = MX1.reshape(_NPAD, 128)[:N]
    cnt_col = cnt[:N, None]
    h1 = _post1(x, xd1, S1, Q1, MN1, MX1, cnt_col, G1, g1)

    xd2, xs2 = _pre_matmuls(h1, W_pre2[:F], W_pre2[F:], b_pre2)
    S2, Q2, MN2, MX2 = _sc_stats_from_parts(xs2, parts, pcnt)
    S2 = S2.reshape(_NPAD, 128)[:N]
    Q2 = Q2.reshape(_NPAD, 128)[:N]
    MN2 = MN2.reshape(_NPAD, 128)[:N]
    MX2 = MX2.reshape(_NPAD, 128)[:N]

    zbar = _post2_zbar(h1, xd2, S2, Q2, MN2, MX2, cnt_col, W_out)[0]
    out = zbar @ G2 + jnp.sum(W_out) * g2 + b_out
    return out[None, :]


# speedup vs baseline: 2.6967x; 1.1118x over previous
import functools
import numpy as np
import jax
import jax.numpy as jnp
from jax import lax
from jax.experimental import pallas as pl
from jax.experimental.pallas import tpu as pltpu
from jax.experimental.pallas import tpu_sc as plsc

N = 10000
E = 320000
F = 128
_AVG_LOG = float(np.log(33.0))

_NW = 32
_NCHUNK = 64
_M = 160
_NPAD = _NCHUNK * _M
_BE = 2000
_NB = E // _BE
_GV = 5
_NG = _BE // (16 * _GV)
_G = 256
_STAGE = _G + 16 * _GV
_ECAP = E + _G

_SC_MESH_KW = dict(
    compiler_params=pltpu.CompilerParams(needs_layout_passes=False),
)


def _acc_outs():
    return (
        jax.ShapeDtypeStruct((_NPAD * 128,), jnp.float32),
        jax.ShapeDtypeStruct((_NPAD * 128,), jnp.float32),
        jax.ShapeDtypeStruct((_NPAD * 128,), jnp.float32),
        jax.ShapeDtypeStruct((_NPAD * 128,), jnp.float32),
    )


def _make_rmw(sstg, rows, accS, accQ, accMN, accMX):
    def rmw(count):
        def edge_body(j, _):
            packed = sstg[pl.ds(j, 16)][0]
            base = (packed >> 18) * 128
            for c in range(8):
                r = rows[j, pl.ds(c * 16, 16)]
                asl = pl.ds(base + c * 16, 16)
                plsc.addupdate(accS.at[asl], r)
                plsc.addupdate(accQ.at[asl], r * r)
                mv = accMN[asl]
                accMN[asl] = jnp.minimum(mv, r)
                xv = accMX[asl]
                accMX[asl] = jnp.maximum(xv, r)
            return 0
        lax.fori_loop(0, count, edge_body, 0)
    return rmw


def _make_fire_gather(xs_hbm, sstg, gidx, rows, sem):
    smask16 = jnp.full((16,), 0x3FFFF, jnp.int32)
    nmax16 = jnp.full((16,), N - 1, jnp.int32)

    def fire_gather():
        for t in range(_G // 16):
            gidx[pl.ds(t * 16, 16)] = jnp.minimum(
                sstg[pl.ds(t * 16, 16)] & smask16, nmax16)
        pltpu.async_copy(xs_hbm.at[gidx], rows, sem).wait()
    return fire_gather


def _make_zero_accs(accS, accQ, accMN, accMX):
    zero16 = jnp.zeros((16,), jnp.float32)
    inf16 = jnp.full((16,), jnp.inf, jnp.float32)

    def zero_accs():
        def zero_body(j, _):
            sl = pl.ds(j * 16, 16)
            accS[sl] = zero16
            accQ[sl] = zero16
            accMN[sl] = inf16
            accMX[sl] = -inf16
            return 0
        lax.fori_loop(0, _M * 128 // 16, zero_body, 0)
    return zero_accs


def _writeback(accS, accQ, accMN, accMX, s_out, q_out, mn_out, mx_out, lo):
    pltpu.sync_copy(accS, s_out.at[pl.ds(lo * 128, _M * 128)])
    pltpu.sync_copy(accQ, q_out.at[pl.ds(lo * 128, _M * 128)])
    pltpu.sync_copy(accMN, mn_out.at[pl.ds(lo * 128, _M * 128)])
    pltpu.sync_copy(accMX, mx_out.at[pl.ds(lo * 128, _M * 128)])


def _sc_stats_partition(xs, src, dst):
    mesh = plsc.VectorSubcoreMesh(core_axis_name="c", subcore_axis_name="s")

    @functools.partial(
        pl.kernel,
        mesh=mesh,
        out_type=_acc_outs() + (
            jax.ShapeDtypeStruct((_NPAD,), jnp.float32),
            jax.ShapeDtypeStruct((_NCHUNK, _ECAP), jnp.int32),
            jax.ShapeDtypeStruct((_NCHUNK * 16,), jnp.int32),
        ),
        scratch_types=[
            pltpu.VMEM((_M * 128,), jnp.float32),
            pltpu.VMEM((_M * 128,), jnp.float32),
            pltpu.VMEM((_M * 128,), jnp.float32),
            pltpu.VMEM((_M * 128,), jnp.float32),
            pltpu.VMEM((_M + 16,), jnp.float32),
            pltpu.VMEM((_BE,), jnp.int32),
            pltpu.VMEM((_BE,), jnp.int32),
            pltpu.VMEM((_BE,), jnp.int32),
            pltpu.VMEM((_BE,), jnp.int32),
            pltpu.VMEM((_STAGE + 32,), jnp.int32),
            pltpu.VMEM((_G,), jnp.int32),
            pltpu.VMEM((_G, 128), jnp.float32),
            pltpu.VMEM((32,), jnp.int32),
            pltpu.VMEM((16,), jnp.int32),
            pltpu.SemaphoreType.DMA,
            pltpu.SemaphoreType.DMA,
            pltpu.SemaphoreType.DMA,
        ],
        **_SC_MESH_KW,
    )
    def k(xs_hbm, src_hbm, dst_hbm, s_out, q_out, mn_out, mx_out, cnt_out,
          parts_out, pcnt_out, accS, accQ, accMN, accMX, cntv, dbufA, sbufA,
          dbufB, sbufB, sstg, gidx, rows, shsc, tebuf, sem, semA, semB):
        wid = lax.axis_index("s") * 2 + lax.axis_index("c")
        zero16 = jnp.zeros((16,), jnp.float32)
        ones16 = jnp.ones((16,), jnp.float32)
        rmw = _make_rmw(sstg, rows, accS, accQ, accMN, accMX)
        fire_gather = _make_fire_gather(xs_hbm, sstg, gidx, rows, sem)
        zero_accs = _make_zero_accs(accS, accQ, accMN, accMX)

        for t in range((_STAGE + 32) // 16):
            sstg[pl.ds(t * 16, 16)] = jnp.zeros((16,), jnp.int32)
        shsc[pl.ds(0, 16)] = jnp.zeros((16,), jnp.int32)

        for ci in range(2):
            chunk = wid * 2 + ci
            lo = chunk * _M
            hi = lo + _M
            zero_accs()
            for t in range((_M + 16) // 16):
                cntv[pl.ds(t * 16, 16)] = zero16

            def drain(carry):
                off, nd = carry
                pltpu.sync_copy(sstg.at[pl.ds(0, _G)],
                                parts_out.at[chunk, pl.ds(nd * _G, _G)])
                fire_gather()
                rmw(_G)
                for t in range(_GV):
                    sstg[pl.ds(t * 16, 16)] = sstg[pl.ds(_G + t * 16, 16)]
                return off - _G, nd + 1

            def filter_batch(dbuf, sbuf, carry):
                def group_body(ig, carry):
                    off, nd = carry
                    gbase = ig * (16 * _GV)
                    for v in range(_GV):
                        sl = pl.ds(gbase + v * 16, 16)
                        d = dbuf[sl]
                        m = (d >= lo) & (d < hi)
                        pc = plsc.all_reduce_population_count(m)[0]

                        def hit(o):
                            s = sbuf[sl]
                            packed = ((d - lo) << 18) | s
                            p = jnp.where(m, 1, 0)
                            for sh in (1, 2, 4, 8):
                                shsc[pl.ds(16, 16)] = p
                                p = p + shsc[pl.ds(16 - sh, 16)]
                            pos = jnp.where(m, p + (o - 1), _STAGE + 16)
                            plsc.store_scatter(sstg, [pos], packed)
                            cpos = jnp.where(m, d - lo, _M)
                            plsc.addupdate_scatter(cntv, [cpos], ones16)
                            return o + pc

                        off = lax.cond(pc > 0, hit, lambda o: o, off)
                    return lax.cond(off >= _G, drain, lambda c: c, (off, nd))

                return lax.fori_loop(0, _NG, group_body, carry)

            def issue(ib, dbuf, sbuf, bsem):
                ebase = jnp.minimum(ib, _NB - 1) * _BE
                pltpu.async_copy(dst_hbm.at[pl.ds(ebase, _BE)], dbuf, bsem)
                pltpu.async_copy(src_hbm.at[pl.ds(ebase, _BE)], sbuf, bsem)

            def wait(dbuf, sbuf, bsem):
                pltpu.make_async_copy(dst_hbm.at[pl.ds(0, _BE)], dbuf, bsem).wait()
                pltpu.make_async_copy(src_hbm.at[pl.ds(0, _BE)], sbuf, bsem).wait()

            issue(0, dbufA, sbufA, semA)

            def batch_pair(i, carry):
                wait(dbufA, sbufA, semA)
                issue(2 * i + 1, dbufB, sbufB, semB)
                carry = filter_batch(dbufA, sbufA, carry)
                wait(dbufB, sbufB, semB)
                issue(2 * i + 2, dbufA, sbufA, semA)
                return filter_batch(dbufB, sbufB, carry)

            off, nd = lax.fori_loop(0, _NB // 2, batch_pair,
                                    (jnp.int32(0), jnp.int32(0)))
            wait(dbufA, sbufA, semA)

            pltpu.sync_copy(sstg.at[pl.ds(0, _G)],
                            parts_out.at[chunk, pl.ds(nd * _G, _G)])
            fire_gather()
            rmw(off)
            te = nd * _G + off
            tebuf[pl.ds(0, 16)] = jnp.full((16,), 1, jnp.int32) * te
            pltpu.sync_copy(tebuf, pcnt_out.at[pl.ds(chunk * 16, 16)])

            _writeback(accS, accQ, accMN, accMX, s_out, q_out, mn_out, mx_out, lo)
            pltpu.sync_copy(cntv.at[pl.ds(0, _M)], cnt_out.at[pl.ds(lo, _M)])

    return k(xs, src, dst)


def _sc_stats_from_parts(xs, parts, pcnt):
    mesh = plsc.VectorSubcoreMesh(core_axis_name="c", subcore_axis_name="s")

    @functools.partial(
        pl.kernel,
        mesh=mesh,
        out_type=_acc_outs(),
        scratch_types=[
            pltpu.VMEM((_M * 128,), jnp.float32),
            pltpu.VMEM((_M * 128,), jnp.float32),
            pltpu.VMEM((_M * 128,), jnp.float32),
            pltpu.VMEM((_M * 128,), jnp.float32),
            pltpu.VMEM((_STAGE + 32,), jnp.int32),
            pltpu.VMEM((_G,), jnp.int32),
            pltpu.VMEM((_G, 128), jnp.float32),
            pltpu.VMEM((_NCHUNK * 16,), jnp.int32),
            pltpu.SemaphoreType.DMA,
        ],
        **_SC_MESH_KW,
    )
    def k(xs_hbm, parts_hbm, pcnt_hbm, s_out, q_out, mn_out, mx_out,
          accS, accQ, accMN, accMX, sstg, gidx, rows, pcv, sem):
        wid = lax.axis_index("s") * 2 + lax.axis_index("c")
        rmw = _make_rmw(sstg, rows, accS, accQ, accMN, accMX)
        fire_gather = _make_fire_gather(xs_hbm, sstg, gidx, rows, sem)
        zero_accs = _make_zero_accs(accS, accQ, accMN, accMX)
        pltpu.sync_copy(pcnt_hbm, pcv)

        for ci in range(2):
            chunk = wid * 2 + ci
            lo = chunk * _M
            zero_accs()
            te = pcv[pl.ds(chunk * 16, 16)][0]
            nfull = te >> 8
            rem = te & (_G - 1)

            def blk(b, _):
                pltpu.sync_copy(parts_hbm.at[chunk, pl.ds(b * _G, _G)],
                                sstg.at[pl.ds(0, _G)])
                fire_gather()
                rmw(_G)
                return 0
            lax.fori_loop(0, nfull, blk, 0)

            pltpu.sync_copy(parts_hbm.at[chunk, pl.ds(nfull * _G, _G)],
                            sstg.at[pl.ds(0, _G)])
            fire_gather()
            rmw(rem)

            _writeback(accS, accQ, accMN, accMX, s_out, q_out, mn_out, mx_out, lo)

    return k(xs, parts, pcnt)


_POST_BLOCK = 1000


def _epilogue(x_blk, xd, S, Q, MN, MX, cnt_col):
    cc = jnp.maximum(cnt_col, 1.0)
    cpos = cnt_col > 0
    mxs = S / cc
    mean = jnp.where(cpos, xd + mxs, 0.0)
    var = jnp.maximum(Q / cc - mxs * mxs, 0.0)
    std = jnp.sqrt(var + 1e-5)
    mn = jnp.where(cpos, xd + MN, 0.0)
    mx = jnp.where(cpos, xd + MX, 0.0)
    agg = jnp.concatenate([mean, mn, mx, std], axis=-1)
    lg = jnp.log(cc + 1.0)
    a_sc = lg / _AVG_LOG
    b_sc = _AVG_LOG / lg
    return jnp.concatenate([x_blk, agg, agg * a_sc, agg * b_sc], axis=1)


def _pre_kernel(x_ref, wd_ref, ws_ref, bd_ref, o1_ref, o2_ref):
    x = x_ref[:]
    o1_ref[:] = jnp.dot(x, wd_ref[:], preferred_element_type=jnp.float32) + bd_ref[:]
    o2_ref[:] = jnp.dot(x, ws_ref[:], preferred_element_type=jnp.float32)


def _pre_matmuls(x, Wd, Ws, bd):
    nb = N // _POST_BLOCK
    return pl.pallas_call(
        _pre_kernel,
        grid=(nb,),
        in_specs=[
            pl.BlockSpec((_POST_BLOCK, F), lambda i: (i, 0)),
            pl.BlockSpec((F, F), lambda i: (0, 0)),
            pl.BlockSpec((F, F), lambda i: (0, 0)),
            pl.BlockSpec((F,), lambda i: (0,)),
        ],
        out_specs=[
            pl.BlockSpec((_POST_BLOCK, F), lambda i: (i, 0)),
            pl.BlockSpec((_POST_BLOCK, F), lambda i: (i, 0)),
        ],
        out_shape=[
            jax.ShapeDtypeStruct((N, F), jnp.float32),
            jax.ShapeDtypeStruct((N, F), jnp.float32),
        ],
    )(x, Wd, Ws, bd)


def _post1_kernel(x_ref, xd_ref, s_ref, q_ref, mn_ref, mx_ref, cnt_ref,
                  wq_ref, bq_ref, wl_ref, bl_ref, o_ref):
    z = _epilogue(x_ref[:], xd_ref[:], s_ref[:], q_ref[:], mn_ref[:],
                  mx_ref[:], cnt_ref[:])
    o = jnp.dot(z, wq_ref[:], preferred_element_type=jnp.float32) + bq_ref[:]
    o_ref[:] = jnp.dot(o, wl_ref[:], preferred_element_type=jnp.float32) + bl_ref[:]


def _post1(x, xd, S, Q, MN, MX, cnt, Wq, bq, Wl, bl):
    nb = N // _POST_BLOCK
    blk = lambda w: pl.BlockSpec((_POST_BLOCK, w), lambda i: (i, 0))
    return pl.pallas_call(
        _post1_kernel,
        grid=(nb,),
        in_specs=[
            blk(F), blk(F), blk(F), blk(F), blk(F), blk(F),
            pl.BlockSpec((_POST_BLOCK, 1), lambda i: (i, 0)),
            pl.BlockSpec((13 * F, F), lambda i: (0, 0)),
            pl.BlockSpec((F,), lambda i: (0,)),
            pl.BlockSpec((F, F), lambda i: (0, 0)),
            pl.BlockSpec((F,), lambda i: (0,)),
        ],
        out_specs=blk(F),
        out_shape=jax.ShapeDtypeStruct((N, F), jnp.float32),
    )(x, xd, S, Q, MN, MX, cnt, Wq, bq, Wl, bl)


def _post2_kernel(h_ref, xd_ref, s_ref, q_ref, mn_ref, mx_ref, cnt_ref,
                  w_ref, o_ref):
    i = pl.program_id(0)
    z = _epilogue(h_ref[:], xd_ref[:], s_ref[:], q_ref[:], mn_ref[:],
                  mx_ref[:], cnt_ref[:])
    part = jnp.dot(w_ref[:].reshape(1, _POST_BLOCK), z,
                   preferred_element_type=jnp.float32,
                   precision=jax.lax.Precision.HIGHEST)

    @pl.when(i == 0)
    def _():
        o_ref[:] = jnp.zeros_like(o_ref)
    o_ref[:] += part


def _post2_zbar(h1, xd, S, Q, MN, MX, cnt, w):
    nb = N // _POST_BLOCK
    blk = lambda wdt: pl.BlockSpec((_POST_BLOCK, wdt), lambda i: (i, 0))
    return pl.pallas_call(
        _post2_kernel,
        grid=(nb,),
        in_specs=[
            blk(F), blk(F), blk(F), blk(F), blk(F), blk(F),
            pl.BlockSpec((_POST_BLOCK, 1), lambda i: (i, 0)),
            pl.BlockSpec((_POST_BLOCK, 1), lambda i: (i, 0)),
        ],
        out_specs=pl.BlockSpec((1, 13 * F), lambda i: (0, 0)),
        out_shape=jax.ShapeDtypeStruct((1, 13 * F), jnp.float32),
    )(h1, xd, S, Q, MN, MX, cnt, w)


def kernel(x, edge_index, edge_weights, batch, W_pre1, b_pre1, W_post1, b_post1,
           W_lin1, b_lin1, W_pre2, b_pre2, W_post2, b_post2, W_lin2, b_lin2,
           W_out, b_out):
    src = edge_index[0]
    dst = edge_index[1]


    xd1, xs1 = _pre_matmuls(x, W_pre1[:F], W_pre1[F:], b_pre1)
    S1, Q1, MN1, MX1, cnt, parts, pcnt = _sc_stats_partition(xs1, src, dst)
    S1 = S1.reshape(_NPAD, 128)[:N]
    Q1 = Q1.reshape(_NPAD, 128)[:N]
    MN1 = MN1.reshape(_NPAD, 128)[:N]
    MX1 = MX1.reshape(_NPAD, 128)[:N]
    cnt_col = cnt[:N, None]
    h1 = jnp.maximum(_post1(x, xd1, S1, Q1, MN1, MX1, cnt_col,
                            W_post1, b_post1, W_lin1, b_lin1), 0.0)

    xd2, xs2 = _pre_matmuls(h1, W_pre2[:F], W_pre2[F:], b_pre2)
    S2, Q2, MN2, MX2 = _sc_stats_from_parts(xs2, parts, pcnt)
    S2 = S2.reshape(_NPAD, 128)[:N]
    Q2 = Q2.reshape(_NPAD, 128)[:N]
    MN2 = MN2.reshape(_NPAD, 128)[:N]
    MX2 = MX2.reshape(_NPAD, 128)[:N]

    h2 = _post1(h1, xd2, S2, Q2, MN2, MX2, cnt_col,
                W_post2, b_post2, W_lin2, b_lin2)
    out = (h2.T @ W_out + b_out).T
    return out


# speedup vs baseline: 2.6967x; 1.0000x over previous
import functools
import numpy as np
import jax
import jax.numpy as jnp
from jax import lax
from jax.experimental import pallas as pl
from jax.experimental.pallas import tpu as pltpu
from jax.experimental.pallas import tpu_sc as plsc

N = 10000
E = 320000
F = 128
_AVG_LOG = float(np.log(33.0))

_NW = 32
_NCHUNK = 64
_M = 160
_NPAD = _NCHUNK * _M
_BE = 2000
_NB = E // _BE
_GV = 5
_NG = _BE // (16 * _GV)
_G = 256
_STAGE = _G + 16 * _GV
_ECAP = E + _G

_SC_MESH_KW = dict(
    compiler_params=pltpu.CompilerParams(needs_layout_passes=False),
)


def _acc_outs():
    return (
        jax.ShapeDtypeStruct((_NPAD * 128,), jnp.float32),
        jax.ShapeDtypeStruct((_NPAD * 128,), jnp.float32),
        jax.ShapeDtypeStruct((_NPAD * 128,), jnp.float32),
        jax.ShapeDtypeStruct((_NPAD * 128,), jnp.float32),
    )


def _make_rmw(sstg, rows, accS, accQ, accMN, accMX):
    def rmw(count):
        def edge_body(j, _):
            packed = sstg[pl.ds(j, 16)][0]
            base = (packed >> 18) * 128
            for c in range(8):
                r = rows[j, pl.ds(c * 16, 16)]
                asl = pl.ds(base + c * 16, 16)
                plsc.addupdate(accS.at[asl], r)
                plsc.addupdate(accQ.at[asl], r * r)
                mv = accMN[asl]
                accMN[asl] = jnp.minimum(mv, r)
                xv = accMX[asl]
                accMX[asl] = jnp.maximum(xv, r)
            return 0
        lax.fori_loop(0, count, edge_body, 0)
    return rmw


def _make_fire_gather(xs_hbm, sstg, gidx, rows, sem):
    smask16 = jnp.full((16,), 0x3FFFF, jnp.int32)
    nmax16 = jnp.full((16,), N - 1, jnp.int32)

    def fire_gather():
        for t in range(_G // 16):
            gidx[pl.ds(t * 16, 16)] = jnp.minimum(
                sstg[pl.ds(t * 16, 16)] & smask16, nmax16)
        pltpu.async_copy(xs_hbm.at[gidx], rows, sem).wait()
    return fire_gather


def _make_zero_accs(accS, accQ, accMN, accMX):
    zero16 = jnp.zeros((16,), jnp.float32)
    inf16 = jnp.full((16,), jnp.inf, jnp.float32)

    def zero_accs():
        def zero_body(j, _):
            sl = pl.ds(j * 16, 16)
            accS[sl] = zero16
            accQ[sl] = zero16
            accMN[sl] = inf16
            accMX[sl] = -inf16
            return 0
        lax.fori_loop(0, _M * 128 // 16, zero_body, 0)
    return zero_accs


def _writeback(accS, accQ, accMN, accMX, s_out, q_out, mn_out, mx_out, lo):
    pltpu.sync_copy(accS, s_out.at[pl.ds(lo * 128, _M * 128)])
    pltpu.sync_copy(accQ, q_out.at[pl.ds(lo * 128, _M * 128)])
    pltpu.sync_copy(accMN, mn_out.at[pl.ds(lo * 128, _M * 128)])
    pltpu.sync_copy(accMX, mx_out.at[pl.ds(lo * 128, _M * 128)])


def _sc_stats_partition(xs, src, dst):
    mesh = plsc.VectorSubcoreMesh(core_axis_name="c", subcore_axis_name="s")

    @functools.partial(
        pl.kernel,
        mesh=mesh,
        out_type=_acc_outs() + (
            jax.ShapeDtypeStruct((_NPAD,), jnp.float32),
            jax.ShapeDtypeStruct((_NCHUNK, _ECAP), jnp.int32),
            jax.ShapeDtypeStruct((_NCHUNK * 16,), jnp.int32),
        ),
        scratch_types=[
            pltpu.VMEM((_M * 128,), jnp.float32),
            pltpu.VMEM((_M * 128,), jnp.float32),
            pltpu.VMEM((_M * 128,), jnp.float32),
            pltpu.VMEM((_M * 128,), jnp.float32),
            pltpu.VMEM((_M + 16,), jnp.float32),
            pltpu.VMEM((_BE,), jnp.int32),
            pltpu.VMEM((_BE,), jnp.int32),
            pltpu.VMEM((_BE,), jnp.int32),
            pltpu.VMEM((_BE,), jnp.int32),
            pltpu.VMEM((_STAGE + 32,), jnp.int32),
            pltpu.VMEM((_G,), jnp.int32),
            pltpu.VMEM((_G, 128), jnp.float32),
            pltpu.VMEM((32,), jnp.int32),
            pltpu.VMEM((16,), jnp.int32),
            pltpu.SemaphoreType.DMA,
            pltpu.SemaphoreType.DMA,
            pltpu.SemaphoreType.DMA,
        ],
        **_SC_MESH_KW,
    )
    def k(xs_hbm, src_hbm, dst_hbm, s_out, q_out, mn_out, mx_out, cnt_out,
          parts_out, pcnt_out, accS, accQ, accMN, accMX, cntv, dbufA, sbufA,
          dbufB, sbufB, sstg, gidx, rows, shsc, tebuf, sem, semA, semB):
        wid = lax.axis_index("s") * 2 + lax.axis_index("c")
        zero16 = jnp.zeros((16,), jnp.float32)
        ones16 = jnp.ones((16,), jnp.float32)
        rmw = _make_rmw(sstg, rows, accS, accQ, accMN, accMX)
        fire_gather = _make_fire_gather(xs_hbm, sstg, gidx, rows, sem)
        zero_accs = _make_zero_accs(accS, accQ, accMN, accMX)

        for t in range((_STAGE + 32) // 16):
            sstg[pl.ds(t * 16, 16)] = jnp.zeros((16,), jnp.int32)
        shsc[pl.ds(0, 16)] = jnp.zeros((16,), jnp.int32)

        for ci in range(2):
            chunk = wid * 2 + ci
            lo = chunk * _M
            hi = lo + _M
            zero_accs()
            for t in range((_M + 16) // 16):
                cntv[pl.ds(t * 16, 16)] = zero16

            def drain(carry):
                off, nd = carry
                pltpu.sync_copy(sstg.at[pl.ds(0, _G)],
                                parts_out.at[chunk, pl.ds(nd * _G, _G)])
                fire_gather()
                rmw(_G)
                for t in range(_GV):
                    sstg[pl.ds(t * 16, 16)] = sstg[pl.ds(_G + t * 16, 16)]
                return off - _G, nd + 1

            def filter_batch(dbuf, sbuf, carry):
                def group_body(ig, carry):
                    off, nd = carry
                    gbase = ig * (16 * _GV)
                    for v in range(_GV):
                        sl = pl.ds(gbase + v * 16, 16)
                        d = dbuf[sl]
                        m = (d >= lo) & (d < hi)
                        pc = plsc.all_reduce_population_count(m)[0]

                        def hit(o):
                            s = sbuf[sl]
                            packed = ((d - lo) << 18) | s
                            p = jnp.where(m, 1, 0)
                            for sh in (1, 2, 4, 8):
                                shsc[pl.ds(16, 16)] = p
                                p = p + shsc[pl.ds(16 - sh, 16)]
                            pos = jnp.where(m, p + (o - 1), _STAGE + 16)
                            plsc.store_scatter(sstg, [pos], packed)
                            cpos = jnp.where(m, d - lo, _M)
                            plsc.addupdate_scatter(cntv, [cpos], ones16)
                            return o + pc

                        off = lax.cond(pc > 0, hit, lambda o: o, off)
                    return lax.cond(off >= _G, drain, lambda c: c, (off, nd))

                return lax.fori_loop(0, _NG, group_body, carry)

            def issue(ib, dbuf, sbuf, bsem):
                ebase = jnp.minimum(ib, _NB - 1) * _BE
                pltpu.async_copy(dst_hbm.at[pl.ds(ebase, _BE)], dbuf, bsem)
                pltpu.async_copy(src_hbm.at[pl.ds(ebase, _BE)], sbuf, bsem)

            def wait(dbuf, sbuf, bsem):
                pltpu.make_async_copy(dst_hbm.at[pl.ds(0, _BE)], dbuf, bsem).wait()
                pltpu.make_async_copy(src_hbm.at[pl.ds(0, _BE)], sbuf, bsem).wait()

            issue(0, dbufA, sbufA, semA)

            def batch_pair(i, carry):
                wait(dbufA, sbufA, semA)
                issue(2 * i + 1, dbufB, sbufB, semB)
                carry = filter_batch(dbufA, sbufA, carry)
                wait(dbufB, sbufB, semB)
                issue(2 * i + 2, dbufA, sbufA, semA)
                return filter_batch(dbufB, sbufB, carry)

            off, nd = lax.fori_loop(0, _NB // 2, batch_pair,
                                    (jnp.int32(0), jnp.int32(0)))
            wait(dbufA, sbufA, semA)

            pltpu.sync_copy(sstg.at[pl.ds(0, _G)],
                            parts_out.at[chunk, pl.ds(nd * _G, _G)])
            fire_gather()
            rmw(off)
            te = nd * _G + off
            tebuf[pl.ds(0, 16)] = jnp.full((16,), 1, jnp.int32) * te
            pltpu.sync_copy(tebuf, pcnt_out.at[pl.ds(chunk * 16, 16)])

            _writeback(accS, accQ, accMN, accMX, s_out, q_out, mn_out, mx_out, lo)
            pltpu.sync_copy(cntv.at[pl.ds(0, _M)], cnt_out.at[pl.ds(lo, _M)])

    return k(xs, src, dst)


def _sc_stats_from_parts(xs, parts, pcnt):
    mesh = plsc.VectorSubcoreMesh(core_axis_name="c", subcore_axis_name="s")

    @functools.partial(
        pl.kernel,
        mesh=mesh,
        out_type=_acc_outs(),
        scratch_types=[
            pltpu.VMEM((_M * 128,), jnp.float32),
            pltpu.VMEM((_M * 128,), jnp.float32),
            pltpu.VMEM((_M * 128,), jnp.float32),
            pltpu.VMEM((_M * 128,), jnp.float32),
            pltpu.VMEM((_STAGE + 32,), jnp.int32),
            pltpu.VMEM((_G,), jnp.int32),
            pltpu.VMEM((_G, 128), jnp.float32),
            pltpu.VMEM((_NCHUNK * 16,), jnp.int32),
            pltpu.SemaphoreType.DMA,
        ],
        **_SC_MESH_KW,
    )
    def k(xs_hbm, parts_hbm, pcnt_hbm, s_out, q_out, mn_out, mx_out,
          accS, accQ, accMN, accMX, sstg, gidx, rows, pcv, sem):
        wid = lax.axis_index("s") * 2 + lax.axis_index("c")
        rmw = _make_rmw(sstg, rows, accS, accQ, accMN, accMX)
        fire_gather = _make_fire_gather(xs_hbm, sstg, gidx, rows, sem)
        zero_accs = _make_zero_accs(accS, accQ, accMN, accMX)
        pltpu.sync_copy(pcnt_hbm, pcv)

        for ci in range(2):
            chunk = wid * 2 + ci
            lo = chunk * _M
            zero_accs()
            te = pcv[pl.ds(chunk * 16, 16)][0]
            nfull = te >> 8
            rem = te & (_G - 1)

            def blk(b, _):
                pltpu.sync_copy(parts_hbm.at[chunk, pl.ds(b * _G, _G)],
                                sstg.at[pl.ds(0, _G)])
                fire_gather()
                rmw(_G)
                return 0
            lax.fori_loop(0, nfull, blk, 0)

            pltpu.sync_copy(parts_hbm.at[chunk, pl.ds(nfull * _G, _G)],
                            sstg.at[pl.ds(0, _G)])
            fire_gather()
            rmw(rem)

            _writeback(accS, accQ, accMN, accMX, s_out, q_out, mn_out, mx_out, lo)

    return k(xs, parts, pcnt)


_POST_BLOCK = 1000


def _epilogue(x_blk, xd, S, Q, MN, MX, cnt_col):
    cc = jnp.maximum(cnt_col, 1.0)
    cpos = cnt_col > 0
    mxs = S / cc
    mean = jnp.where(cpos, xd + mxs, 0.0)
    var = jnp.maximum(Q / cc - mxs * mxs, 0.0)
    std = jnp.sqrt(var + 1e-5)
    mn = jnp.where(cpos, xd + MN, 0.0)
    mx = jnp.where(cpos, xd + MX, 0.0)
    agg = jnp.concatenate([mean, mn, mx, std], axis=-1)
    lg = jnp.log(cc + 1.0)
    a_sc = lg / _AVG_LOG
    b_sc = _AVG_LOG / lg
    return jnp.concatenate([x_blk, agg, agg * a_sc, agg * b_sc], axis=1)


def _pre_kernel(x_ref, wd_ref, ws_ref, bd_ref, o1_ref, o2_ref):
    x = x_ref[:]
    o1_ref[:] = jnp.dot(x, wd_ref[:], preferred_element_type=jnp.float32) + bd_ref[:]
    o2_ref[:] = jnp.dot(x, ws_ref[:], preferred_element_type=jnp.float32)


def _pre_matmuls(x, Wd, Ws, bd):
    nb = N // _POST_BLOCK
    return pl.pallas_call(
        _pre_kernel,
        grid=(nb,),
        in_specs=[
            pl.BlockSpec((_POST_BLOCK, F), lambda i: (i, 0)),
            pl.BlockSpec((F, F), lambda i: (0, 0)),
            pl.BlockSpec((F, F), lambda i: (0, 0)),
            pl.BlockSpec((F,), lambda i: (0,)),
        ],
        out_specs=[
            pl.BlockSpec((_POST_BLOCK, F), lambda i: (i, 0)),
            pl.BlockSpec((_POST_BLOCK, F), lambda i: (i, 0)),
        ],
        out_shape=[
            jax.ShapeDtypeStruct((N, F), jnp.float32),
            jax.ShapeDtypeStruct((N, F), jnp.float32),
        ],
    )(x, Wd, Ws, bd)


def _post1_kernel(x_ref, xd_ref, s_ref, q_ref, mn_ref, mx_ref, cnt_ref,
                  wq_ref, bq_ref, wl_ref, bl_ref, o_ref):
    z = _epilogue(x_ref[:], xd_ref[:], s_ref[:], q_ref[:], mn_ref[:],
                  mx_ref[:], cnt_ref[:])
    o = jnp.dot(z, wq_ref[:], preferred_element_type=jnp.float32) + bq_ref[:]
    o_ref[:] = jnp.dot(o, wl_ref[:], preferred_element_type=jnp.float32) + bl_ref[:]


def _post1(x, xd, S, Q, MN, MX, cnt, Wq, bq, Wl, bl):
    nb = N // _POST_BLOCK
    blk = lambda w: pl.BlockSpec((_POST_BLOCK, w), lambda i: (i, 0))
    return pl.pallas_call(
        _post1_kernel,
        grid=(nb,),
        in_specs=[
            blk(F), blk(F), blk(F), blk(F), blk(F), blk(F),
            pl.BlockSpec((_POST_BLOCK, 1), lambda i: (i, 0)),
            pl.BlockSpec((13 * F, F), lambda i: (0, 0)),
            pl.BlockSpec((F,), lambda i: (0,)),
            pl.BlockSpec((F, F), lambda i: (0, 0)),
            pl.BlockSpec((F,), lambda i: (0,)),
        ],
        out_specs=blk(F),
        out_shape=jax.ShapeDtypeStruct((N, F), jnp.float32),
    )(x, xd, S, Q, MN, MX, cnt, Wq, bq, Wl, bl)


def kernel(x, edge_index, edge_weights, batch, W_pre1, b_pre1, W_post1, b_post1,
           W_lin1, b_lin1, W_pre2, b_pre2, W_post2, b_post2, W_lin2, b_lin2,
           W_out, b_out):
    src = edge_index[0]
    dst = edge_index[1]


    xd1, xs1 = _pre_matmuls(x, W_pre1[:F], W_pre1[F:], b_pre1)
    S1, Q1, MN1, MX1, cnt, parts, pcnt = _sc_stats_partition(xs1, src, dst)
    S1 = S1.reshape(_NPAD, 128)[:N]
    Q1 = Q1.reshape(_NPAD, 128)[:N]
    MN1 = MN1.reshape(_NPAD, 128)[:N]
    MX1 = MX1.reshape(_NPAD, 128)[:N]
    cnt_col = cnt[:N, None]
    h1 = jnp.maximum(_post1(x, xd1, S1, Q1, MN1, MX1, cnt_col,
                            W_post1, b_post1, W_lin1, b_lin1), 0.0)

    xd2, xs2 = _pre_matmuls(h1, W_pre2[:F], W_pre2[F:], b_pre2)
    S2, Q2, MN2, MX2 = _sc_stats_from_parts(xs2, parts, pcnt)
    S2 = S2.reshape(_NPAD, 128)[:N]
    Q2 = Q2.reshape(_NPAD, 128)[:N]
    MN2 = MN2.reshape(_NPAD, 128)[:N]
    MX2 = MX2.reshape(_NPAD, 128)[:N]

    h2 = _post1(h1, xd2, S2, Q2, MN2, MX2, cnt_col,
                W_post2, b_post2, W_lin2, b_lin2)
    out = (h2.T @ W_out + b_out).T
    return out
